# Initial kernel scaffold; baseline (speedup 1.0000x reference)
import jax
import jax.numpy as jnp
from jax.experimental import pallas as pl


def kernel(cls_logits, bbox_pred_cxcywh, gt_boxes_batch, gt_labels_batch, default_boxes_xyxy):
    raise NotImplementedError("write your pallas kernel here")


# speedup vs baseline: 14.1273x; 14.1273x over previous
import jax
import jax.numpy as jnp
from jax.experimental import pallas as pl
from jax.experimental.pallas import tpu as pltpu

IOU_POS = 0.5
IOU_NEG = 0.4
NEG_POS_RATIO = 3


def _smooth_l1(d):
    ad = jnp.abs(d)
    return jnp.where(ad < 1.0, 0.5 * d * d, ad - 0.5)


def _loss_kernel(gt_ref, lab_ref, cls_ref, pred_ref, anch_ref, out_ref):
    i = pl.program_id(0)
    B = pl.num_programs(0)
    logits = cls_ref[0]
    N, C = logits.shape

    ax0 = anch_ref[0, :]
    ay0 = anch_ref[1, :]
    ax1 = anch_ref[2, :]
    ay1 = anch_ref[3, :]
    area_a = (ax1 - ax0) * (ay1 - ay0)

    max_iou = jnp.full((N,), -1.0, jnp.float32)
    mx0 = jnp.zeros((N,), jnp.float32)
    my0 = jnp.zeros((N,), jnp.float32)
    mx1 = jnp.zeros((N,), jnp.float32)
    my1 = jnp.zeros((N,), jnp.float32)
    mlab = jnp.zeros((N,), jnp.int32)
    G = gt_ref.shape[1]
    for g in range(G):
        bx0 = gt_ref[i, g, 0]
        by0 = gt_ref[i, g, 1]
        bx1 = gt_ref[i, g, 2]
        by1 = gt_ref[i, g, 3]
        lg = lab_ref[i, g]
        w = jnp.maximum(jnp.minimum(bx1, ax1) - jnp.maximum(bx0, ax0), 0.0)
        h = jnp.maximum(jnp.minimum(by1, ay1) - jnp.maximum(by0, ay0), 0.0)
        inter = w * h
        area_b = (bx1 - bx0) * (by1 - by0)
        union = jnp.maximum(area_a + area_b - inter, 1e-9)
        iou_g = inter / union
        upd = iou_g > max_iou
        max_iou = jnp.where(upd, iou_g, max_iou)
        mx0 = jnp.where(upd, bx0, mx0)
        my0 = jnp.where(upd, by0, my0)
        mx1 = jnp.where(upd, bx1, mx1)
        my1 = jnp.where(upd, by1, my1)
        mlab = jnp.where(upd, lg, mlab)

    pos = max_iou >= IOU_POS
    neg = max_iou < IOU_NEG
    npos_f = jnp.sum(jnp.where(pos, 1.0, 0.0))
    nneg_f = jnp.sum(jnp.where(neg, 1.0, 0.0))

    cx = pred_ref[0, 0, :]
    cy = pred_ref[0, 1, :]
    pw = pred_ref[0, 2, :]
    ph = pred_ref[0, 3, :]
    l0 = _smooth_l1(cx - 0.5 * pw - mx0)
    l1 = _smooth_l1(cy - 0.5 * ph - my0)
    l2 = _smooth_l1(cx + 0.5 * pw - mx1)
    l3 = _smooth_l1(cy + 0.5 * ph - my1)
    loc_i = jnp.sum(jnp.where(pos, l0 + l1 + l2 + l3, 0.0))

    m = jnp.max(logits, axis=-1)
    lse = m + jnp.log(jnp.sum(jnp.exp(logits - m[:, None]), axis=-1))
    tgt = jnp.where(pos, mlab + 1, 0)
    cids = jax.lax.broadcasted_iota(jnp.int32, (N, C), 1)
    lt = jnp.sum(jnp.where(cids == tgt[:, None], logits, 0.0), axis=-1)
    ce = lse - lt
    ce_pos = jnp.sum(jnp.where(pos, ce, 0.0))

    ncap = float(int(N * 0.05))
    k_f = jnp.where(
        npos_f > 0.0,
        jnp.minimum(NEG_POS_RATIO * npos_f, nneg_f),
        jnp.where(nneg_f > 0.0, jnp.minimum(ncap, nneg_f), 0.0),
    )
    ce_neg = jnp.where(neg, ce, -1.0)
    maxv = jnp.maximum(jnp.max(ce_neg), 0.0)
    hi0 = jax.lax.bitcast_convert_type(maxv, jnp.int32)

    def count_gt(t):
        return jnp.sum(jnp.where(ce_neg > t, 1.0, 0.0))

    def bs_body(_, carry):
        lo, hi = carry
        mid = lo + jax.lax.div(hi - lo, 2)
        c = count_gt(jax.lax.bitcast_convert_type(mid, jnp.float32))
        under = c < k_f
        return (jnp.where(under, lo, mid + 1), jnp.where(under, mid, hi))

    lo, _ = jax.lax.fori_loop(0, 31, bs_body, (jnp.int32(0), hi0))
    tstar = jax.lax.bitcast_convert_type(lo, jnp.float32)
    cgt = count_gt(tstar)
    sum_gt = jnp.sum(jnp.where(ce_neg > tstar, ce_neg, 0.0))
    hard = sum_gt + (k_f - cgt) * tstar
    hard = jnp.where(k_f > 0.0, hard, 0.0)
    conf_i = ce_pos + hard

    @pl.when(i == 0)
    def _init():
        out_ref[0] = 0.0
        out_ref[1] = 0.0
        out_ref[2] = 0.0

    out_ref[0] += loc_i
    out_ref[1] += conf_i
    out_ref[2] += npos_f

    @pl.when(i == B - 1)
    def _finalize():
        ls = out_ref[0]
        cs = out_ref[1]
        tp = out_ref[2]
        zero = tp == 0.0
        loc = jnp.where(zero, 0.0, ls / jnp.maximum(tp, 1.0))
        conf = cs / jnp.where(zero, float(B * N), tp)
        out_ref[0] = loc + conf
        out_ref[1] = loc
        out_ref[2] = conf


def kernel(cls_logits, bbox_pred_cxcywh, gt_boxes_batch, gt_labels_batch, default_boxes_xyxy):
    B, N, C = cls_logits.shape
    G = gt_boxes_batch.shape[1]
    pred_t = jnp.transpose(bbox_pred_cxcywh, (0, 2, 1))
    anch_t = jnp.transpose(default_boxes_xyxy, (1, 0))
    out = pl.pallas_call(
        _loss_kernel,
        grid=(B,),
        in_specs=[
            pl.BlockSpec(memory_space=pltpu.SMEM),
            pl.BlockSpec(memory_space=pltpu.SMEM),
            pl.BlockSpec((1, N, C), lambda i: (i, 0, 0)),
            pl.BlockSpec((1, 4, N), lambda i: (i, 0, 0)),
            pl.BlockSpec((4, N), lambda i: (0, 0)),
        ],
        out_specs=pl.BlockSpec(memory_space=pltpu.SMEM),
        out_shape=jax.ShapeDtypeStruct((3,), jnp.float32),
    )(gt_boxes_batch, gt_labels_batch.astype(jnp.int32), cls_logits, pred_t, anch_t)
    return (out[0], out[1], out[2])


# speedup vs baseline: 23.2477x; 1.6456x over previous
import jax
import jax.numpy as jnp
from jax.experimental import pallas as pl
from jax.experimental.pallas import tpu as pltpu

IOU_POS = 0.5
IOU_NEG = 0.4
NEG_POS_RATIO = 3


def _smooth_l1(d):
    ad = jnp.abs(d)
    return jnp.where(ad < 1.0, 0.5 * d * d, ad - 0.5)


def _loss_kernel(gt_ref, lab_ref, cls_ref, pred_ref, anch_ref, out_ref):
    i = pl.program_id(0)
    B = pl.num_programs(0)
    logits = cls_ref[0]
    C, N = logits.shape

    ax0 = anch_ref[0, :]
    ay0 = anch_ref[1, :]
    ax1 = anch_ref[2, :]
    ay1 = anch_ref[3, :]
    area_a = (ax1 - ax0) * (ay1 - ay0)

    max_iou = jnp.full((N,), -1.0, jnp.float32)
    mx0 = jnp.zeros((N,), jnp.float32)
    my0 = jnp.zeros((N,), jnp.float32)
    mx1 = jnp.zeros((N,), jnp.float32)
    my1 = jnp.zeros((N,), jnp.float32)
    mlab = jnp.zeros((N,), jnp.int32)
    G = gt_ref.shape[1]
    for g in range(G):
        bx0 = gt_ref[i, g, 0]
        by0 = gt_ref[i, g, 1]
        bx1 = gt_ref[i, g, 2]
        by1 = gt_ref[i, g, 3]
        lg = lab_ref[i, g]
        w = jnp.maximum(jnp.minimum(bx1, ax1) - jnp.maximum(bx0, ax0), 0.0)
        h = jnp.maximum(jnp.minimum(by1, ay1) - jnp.maximum(by0, ay0), 0.0)
        inter = w * h
        area_b = (bx1 - bx0) * (by1 - by0)
        union = jnp.maximum(area_a + area_b - inter, 1e-9)
        iou_g = inter / union
        upd = iou_g > max_iou
        max_iou = jnp.where(upd, iou_g, max_iou)
        mx0 = jnp.where(upd, bx0, mx0)
        my0 = jnp.where(upd, by0, my0)
        mx1 = jnp.where(upd, bx1, mx1)
        my1 = jnp.where(upd, by1, my1)
        mlab = jnp.where(upd, lg, mlab)

    pos = max_iou >= IOU_POS
    neg = max_iou < IOU_NEG
    npos_f = jnp.sum(jnp.where(pos, 1.0, 0.0))
    nneg_f = jnp.sum(jnp.where(neg, 1.0, 0.0))

    cx = pred_ref[0, 0, :]
    cy = pred_ref[0, 1, :]
    pw = pred_ref[0, 2, :]
    ph = pred_ref[0, 3, :]
    l0 = _smooth_l1(cx - 0.5 * pw - mx0)
    l1 = _smooth_l1(cy - 0.5 * ph - my0)
    l2 = _smooth_l1(cx + 0.5 * pw - mx1)
    l3 = _smooth_l1(cy + 0.5 * ph - my1)
    loc_i = jnp.sum(jnp.where(pos, l0 + l1 + l2 + l3, 0.0))

    m = jnp.max(logits, axis=0)
    lse = m + jnp.log(jnp.sum(jnp.exp(logits - m[None, :]), axis=0))
    tgt = jnp.where(pos, mlab + 1, 0)
    cids = jax.lax.broadcasted_iota(jnp.int32, (C, N), 0)
    lt = jnp.sum(jnp.where(cids == tgt[None, :], logits, 0.0), axis=0)
    ce = lse - lt
    ce_pos = jnp.sum(jnp.where(pos, ce, 0.0))

    ncap = float(int(N * 0.05))
    k_f = jnp.where(
        npos_f > 0.0,
        jnp.minimum(NEG_POS_RATIO * npos_f, nneg_f),
        jnp.where(nneg_f > 0.0, jnp.minimum(ncap, nneg_f), 0.0),
    )
    ce_neg = jnp.where(neg, ce, -1.0)
    maxv = jnp.maximum(jnp.max(ce_neg), 0.0)
    hi0 = jax.lax.bitcast_convert_type(maxv, jnp.int32)

    def count_gt(t):
        return jnp.sum(jnp.where(ce_neg > t, 1.0, 0.0))

    def bs_body(_, carry):
        lo, hi = carry
        mid = lo + jax.lax.div(hi - lo, 2)
        c = count_gt(jax.lax.bitcast_convert_type(mid, jnp.float32))
        under = c < k_f
        return (jnp.where(under, lo, mid + 1), jnp.where(under, mid, hi))

    lo, _ = jax.lax.fori_loop(0, 31, bs_body, (jnp.int32(0), hi0))
    tstar = jax.lax.bitcast_convert_type(lo, jnp.float32)
    cgt = count_gt(tstar)
    sum_gt = jnp.sum(jnp.where(ce_neg > tstar, ce_neg, 0.0))
    hard = sum_gt + (k_f - cgt) * tstar
    hard = jnp.where(k_f > 0.0, hard, 0.0)
    conf_i = ce_pos + hard

    @pl.when(i == 0)
    def _init():
        out_ref[0] = 0.0
        out_ref[1] = 0.0
        out_ref[2] = 0.0

    out_ref[0] += loc_i
    out_ref[1] += conf_i
    out_ref[2] += npos_f

    @pl.when(i == B - 1)
    def _finalize():
        ls = out_ref[0]
        cs = out_ref[1]
        tp = out_ref[2]
        zero = tp == 0.0
        loc = jnp.where(zero, 0.0, ls / jnp.maximum(tp, 1.0))
        conf = cs / jnp.where(zero, float(B * N), tp)
        out_ref[0] = loc + conf
        out_ref[1] = loc
        out_ref[2] = conf


def kernel(cls_logits, bbox_pred_cxcywh, gt_boxes_batch, gt_labels_batch, default_boxes_xyxy):
    B, N, C = cls_logits.shape
    G = gt_boxes_batch.shape[1]
    cls_t = jnp.transpose(cls_logits, (0, 2, 1))
    pred_t = jnp.transpose(bbox_pred_cxcywh, (0, 2, 1))
    anch_t = jnp.transpose(default_boxes_xyxy, (1, 0))
    out = pl.pallas_call(
        _loss_kernel,
        grid=(B,),
        in_specs=[
            pl.BlockSpec(memory_space=pltpu.SMEM),
            pl.BlockSpec(memory_space=pltpu.SMEM),
            pl.BlockSpec((1, C, N), lambda i: (i, 0, 0)),
            pl.BlockSpec((1, 4, N), lambda i: (i, 0, 0)),
            pl.BlockSpec((4, N), lambda i: (0, 0)),
        ],
        out_specs=pl.BlockSpec(memory_space=pltpu.SMEM),
        out_shape=jax.ShapeDtypeStruct((3,), jnp.float32),
    )(gt_boxes_batch, gt_labels_batch.astype(jnp.int32), cls_t, pred_t, anch_t)
    return (out[0], out[1], out[2])
